# baseline (device time: 13848 ns/iter reference)
import jax
import jax.numpy as jnp
from jax import lax
from jax.experimental import pallas as pl
from jax.experimental.pallas import tpu as pltpu

C = 2


def kernel(A, B):
    m, k = A.shape
    _, n = B.shape
    nh = n // 2
    nc = nh // C

    def body(a_ref, b_ref, out_ref, xsend, xrecv, ysend, yrecv,
             xsend_sems, xrecv_sems, ysend_sems, yrecv_sems):
        my_x = lax.axis_index("x")
        my_y = lax.axis_index("y")
        xpeer = (1 - my_x, my_y)
        ypeer = (my_x, 1 - my_y)

        barrier_sem = pltpu.get_barrier_semaphore()
        for p in (xpeer, ypeer):
            pl.semaphore_signal(
                barrier_sem, inc=1, device_id=p,
                device_id_type=pl.DeviceIdType.MESH,
            )
        pl.semaphore_wait(barrier_sem, 2)

        a = a_ref[...].astype(jnp.bfloat16)
        col0 = my_y * nh

        x_rdmas = []
        for c in range(C):
            bcol = b_ref[:, pl.ds(col0 + c * nc, nc)].astype(jnp.bfloat16)
            pc = jnp.dot(a, bcol, preferred_element_type=jnp.float32)
            out_ref[:, pl.ds(col0 + c * nc, nc)] = pc
            xsend[c] = pc.astype(jnp.bfloat16)
            r = pltpu.make_async_remote_copy(
                src_ref=xsend.at[c], dst_ref=xrecv.at[c],
                send_sem=xsend_sems.at[c], recv_sem=xrecv_sems.at[c],
                device_id=xpeer, device_id_type=pl.DeviceIdType.MESH,
            )
            r.start()
            x_rdmas.append(r)

        y_rdmas = []
        for c in range(C):
            x_rdmas[c].wait()
            red = out_ref[:, pl.ds(col0 + c * nc, nc)] + xrecv[c].astype(
                jnp.float32
            )
            out_ref[:, pl.ds(col0 + c * nc, nc)] = red
            ysend[c] = red.astype(jnp.bfloat16)
            r = pltpu.make_async_remote_copy(
                src_ref=ysend.at[c], dst_ref=yrecv.at[c],
                send_sem=ysend_sems.at[c], recv_sem=yrecv_sems.at[c],
                device_id=ypeer, device_id_type=pl.DeviceIdType.MESH,
            )
            r.start()
            y_rdmas.append(r)

        ocol0 = (1 - my_y) * nh
        for c in range(C):
            y_rdmas[c].wait()
            out_ref[:, pl.ds(ocol0 + c * nc, nc)] = yrecv[c].astype(
                jnp.float32
            )

    return pl.pallas_call(
        body,
        out_shape=jax.ShapeDtypeStruct((m, n), jnp.float32),
        in_specs=[
            pl.BlockSpec(memory_space=pltpu.VMEM),
            pl.BlockSpec(memory_space=pltpu.VMEM),
        ],
        out_specs=pl.BlockSpec(memory_space=pltpu.VMEM),
        scratch_shapes=[
            pltpu.VMEM((C, m, nc), jnp.bfloat16),
            pltpu.VMEM((C, m, nc), jnp.bfloat16),
            pltpu.VMEM((C, m, nc), jnp.bfloat16),
            pltpu.VMEM((C, m, nc), jnp.bfloat16),
            pltpu.SemaphoreType.DMA((C,)),
            pltpu.SemaphoreType.DMA((C,)),
            pltpu.SemaphoreType.DMA((C,)),
            pltpu.SemaphoreType.DMA((C,)),
        ],
        compiler_params=pltpu.CompilerParams(collective_id=0),
    )(A, B)


# device time: 12770 ns/iter; 1.0844x vs baseline; 1.0844x over previous
import jax
import jax.numpy as jnp
from jax import lax
from jax.experimental import pallas as pl
from jax.experimental.pallas import tpu as pltpu

C = 4


def kernel(A, B):
    m, k = A.shape
    _, n = B.shape
    nc = n // C

    def body(a_ref, b_ref, out_ref, send, recv, send_sems, recv_sems):
        my_x = lax.axis_index("x")
        my_y = lax.axis_index("y")
        peer = (1 - my_x, my_y)

        barrier_sem = pltpu.get_barrier_semaphore()
        pl.semaphore_signal(
            barrier_sem, inc=1, device_id=peer,
            device_id_type=pl.DeviceIdType.MESH,
        )
        pl.semaphore_wait(barrier_sem, 1)

        a = a_ref[...].astype(jnp.bfloat16)

        rdmas = []
        for c in range(C):
            bcol = b_ref[:, c * nc:(c + 1) * nc].astype(jnp.bfloat16)
            pc = jnp.dot(a, bcol, preferred_element_type=jnp.float32)
            out_ref[:, c * nc:(c + 1) * nc] = pc
            send[c] = pc.astype(jnp.bfloat16)
            r = pltpu.make_async_remote_copy(
                src_ref=send.at[c], dst_ref=recv.at[c],
                send_sem=send_sems.at[c], recv_sem=recv_sems.at[c],
                device_id=peer, device_id_type=pl.DeviceIdType.MESH,
            )
            r.start()
            rdmas.append(r)

        for c in range(C):
            rdmas[c].wait()
            out_ref[:, c * nc:(c + 1) * nc] += recv[c].astype(jnp.float32)

    return pl.pallas_call(
        body,
        out_shape=jax.ShapeDtypeStruct((m, n), jnp.float32),
        in_specs=[
            pl.BlockSpec(memory_space=pltpu.VMEM),
            pl.BlockSpec(memory_space=pltpu.VMEM),
        ],
        out_specs=pl.BlockSpec(memory_space=pltpu.VMEM),
        scratch_shapes=[
            pltpu.VMEM((C, m, nc), jnp.bfloat16),
            pltpu.VMEM((C, m, nc), jnp.bfloat16),
            pltpu.SemaphoreType.DMA((C,)),
            pltpu.SemaphoreType.DMA((C,)),
        ],
        compiler_params=pltpu.CompilerParams(collective_id=0),
    )(A, B)


# device time: 12593 ns/iter; 1.0997x vs baseline; 1.0141x over previous
import jax
import jax.numpy as jnp
from jax import lax
from jax.experimental import pallas as pl
from jax.experimental.pallas import tpu as pltpu

C = 4


def kernel(A, B):
    m, k = A.shape
    _, n = B.shape
    nc = n // C

    def body(a_ref, b_ref, out_ref, acc, send, recv, send_sems, recv_sems):
        my_x = lax.axis_index("x")
        my_y = lax.axis_index("y")
        peer = (1 - my_x, my_y)

        barrier_sem = pltpu.get_barrier_semaphore()
        pl.semaphore_signal(
            barrier_sem, inc=1, device_id=peer,
            device_id_type=pl.DeviceIdType.MESH,
        )
        pl.semaphore_wait(barrier_sem, 1)

        a = a_ref[...].astype(jnp.bfloat16)

        rdmas = []
        for c in range(C):
            bcol = b_ref[:, c * nc:(c + 1) * nc].astype(jnp.bfloat16)
            pc = jnp.dot(a, bcol, preferred_element_type=jnp.float32)
            acc[c] = pc
            send[c] = pc.astype(jnp.bfloat16)
            r = pltpu.make_async_remote_copy(
                src_ref=send.at[c], dst_ref=recv.at[c],
                send_sem=send_sems.at[c], recv_sem=recv_sems.at[c],
                device_id=peer, device_id_type=pl.DeviceIdType.MESH,
            )
            r.start()
            rdmas.append(r)

        for c in range(C):
            rdmas[c].wait()
            out_ref[:, c * nc:(c + 1) * nc] = (
                acc[c] + recv[c].astype(jnp.float32)
            ).astype(jnp.bfloat16)

    return pl.pallas_call(
        body,
        out_shape=jax.ShapeDtypeStruct((m, n), jnp.bfloat16),
        in_specs=[
            pl.BlockSpec(memory_space=pltpu.VMEM),
            pl.BlockSpec(memory_space=pltpu.VMEM),
        ],
        out_specs=pl.BlockSpec(memory_space=pltpu.VMEM),
        scratch_shapes=[
            pltpu.VMEM((C, m, nc), jnp.float32),
            pltpu.VMEM((C, m, nc), jnp.bfloat16),
            pltpu.VMEM((C, m, nc), jnp.bfloat16),
            pltpu.SemaphoreType.DMA((C,)),
            pltpu.SemaphoreType.DMA((C,)),
        ],
        compiler_params=pltpu.CompilerParams(collective_id=0),
    )(A, B)
